# baseline (speedup 1.0000x reference)
import functools

import jax
import jax.numpy as jnp
from jax import lax
from jax.experimental import pallas as pl
from jax.experimental.pallas import tpu as pltpu
from jax.experimental.pallas import tpu_sc as plsc

NG = 8192
D = 32
NS = 16384
BETA = 0.99
WIDTH = 128
XB = 512
NW = 32
ROWS_PER_W = NS // NW
CHUNK = 128
NCHUNK = ROWS_PER_W // CHUNK
STRIPE = NG // 16


GB = 128
NGC = NG // GB


def _normalize_body(a_ref, out_ref):
    a = a_ref[...]
    n = jnp.sqrt(jnp.sum(a * a, axis=1, keepdims=True))
    out_ref[...] = a / jnp.maximum(n, 1e-12)


def _normalize_rows(a):
    rows = a.shape[0]
    blk = rows // 4
    return pl.pallas_call(
        _normalize_body,
        grid=(4,),
        in_specs=[pl.BlockSpec((blk, a.shape[1]), lambda i: (i, 0))],
        out_specs=pl.BlockSpec((blk, a.shape[1]), lambda i: (i, 0)),
        out_shape=jax.ShapeDtypeStruct(a.shape, jnp.float32),
    )(a)


def _assign_body(x_ref, gf_ref, out_ref, xaug_ref, gfn_ref):
    i = pl.program_id(0)

    @pl.when(i == 0)
    def _():
        gf = gf_ref[...]
        nn = jnp.sqrt(jnp.sum(gf * gf, axis=1, keepdims=True))
        gfn_ref[...] = gf / jnp.maximum(nn, 1e-12)

    x = x_ref[...]
    xaug_ref[...] = jnp.concatenate(
        [x, jnp.ones((XB, 1), jnp.float32),
         jnp.zeros((XB, WIDTH - D - 1), jnp.float32)], axis=1)
    n = jnp.sqrt(jnp.sum(x * x, axis=1, keepdims=True))
    xn = x / jnp.maximum(n, 1e-12)

    m_run = jnp.full((XB, GB), -jnp.inf, jnp.float32)
    c_run = jnp.zeros((XB, GB), jnp.float32)
    for c in range(NGC):
        chunk = lax.dot_general(
            xn, gfn_ref[c * GB:(c + 1) * GB, :], (((1,), (1,)), ((), ())),
            preferred_element_type=jnp.float32)
        better = chunk > m_run
        c_run = jnp.where(better, jnp.float32(c), c_run)
        m_run = jnp.maximum(chunk, m_run)

    m = jnp.max(m_run, axis=-1, keepdims=True)
    lane = lax.broadcasted_iota(jnp.int32, (XB, GB), 1).astype(jnp.float32)
    j = c_run * GB + lane
    loc = jnp.min(jnp.where(m_run == m, j, jnp.float32(NG)), axis=-1)
    out_ref[0] = loc.astype(jnp.int32).reshape(XB // CHUNK, CHUNK)


def _assign(x, gf):
    grid = x.shape[0] // XB
    rpb = XB // CHUNK
    return pl.pallas_call(
        _assign_body,
        grid=(grid,),
        in_specs=[
            pl.BlockSpec((XB, D), lambda i: (i, 0)),
            pl.BlockSpec((NG, D), lambda i: (0, 0)),
        ],
        out_specs=[
            pl.BlockSpec((1, rpb, CHUNK), lambda i: (i, 0, 0)),
            pl.BlockSpec((XB, WIDTH), lambda i: (i, 0)),
        ],
        out_shape=[
            jax.ShapeDtypeStruct((grid, rpb, CHUNK), jnp.int32),
            jax.ShapeDtypeStruct((x.shape[0], WIDTH), jnp.float32),
        ],
        scratch_shapes=[pltpu.VMEM((NG, D), jnp.float32)],
    )(x, gf)


@functools.lru_cache(maxsize=4)
def _make_scatter(nchunk):
    mesh = plsc.VectorSubcoreMesh(core_axis_name="c", subcore_axis_name="s")
    rows_per_w = nchunk * CHUNK

    @functools.partial(
        pl.kernel,
        mesh=mesh,
        out_type=jax.ShapeDtypeStruct((2 * NG, WIDTH), jnp.float32),
        scratch_types=[
            pltpu.VMEM((nchunk, CHUNK), jnp.int32),
            pltpu.VMEM((CHUNK, WIDTH), jnp.float32),
            pltpu.VMEM((CHUNK, WIDTH), jnp.float32),
            pltpu.SemaphoreType.DMA,
            pltpu.SemaphoreType.DMA,
            pltpu.VMEM_SHARED((NG, WIDTH), jnp.float32),
        ],
    )
    def scatter(xaug_hbm, asn_hbm, out_hbm, idx_v, buf0, buf1, sem0, sem1,
                table_sh):
        NCHUNK = nchunk
        c = lax.axis_index("c")
        s = lax.axis_index("s")
        wid = s * 2 + c
        base = wid * rows_per_w
        bufs = (buf0, buf1)
        sems = (sem0, sem1)

        zv = jnp.zeros((16,), jnp.float32)

        def zrow(r, carry):
            for k in range(WIDTH // 16):
                buf0[r, pl.ds(k * 16, 16)] = zv
            return carry

        lax.fori_loop(0, CHUNK, zrow, 0)
        for q in range(STRIPE // CHUNK):
            pltpu.sync_copy(buf0, table_sh.at[pl.ds(s * STRIPE + q * CHUNK, CHUNK)])
        pltpu.sync_copy(asn_hbm.at[wid], idx_v)
        plsc.subcore_barrier()

        copies = [None] * NCHUNK
        copies[0] = pltpu.async_copy(
            xaug_hbm.at[pl.ds(base, CHUNK)], bufs[0], sems[0])
        for q in range(NCHUNK):
            copies[q].wait()
            if q + 1 < NCHUNK:
                copies[q + 1] = pltpu.async_copy(
                    xaug_hbm.at[pl.ds(base + (q + 1) * CHUNK, CHUNK)],
                    bufs[(q + 1) % 2], sems[(q + 1) % 2])
            pltpu.sync_copy(bufs[q % 2], table_sh.at[idx_v.at[q]], add=True)
        plsc.subcore_barrier()

        nst = STRIPE // CHUNK
        stores = [None] * nst
        for q in range(nst):
            if q >= 2:
                stores[q - 2].wait()
            pltpu.sync_copy(table_sh.at[pl.ds(s * STRIPE + q * CHUNK, CHUNK)],
                            bufs[q % 2])
            stores[q] = pltpu.async_copy(
                bufs[q % 2],
                out_hbm.at[pl.ds(c * NG + s * STRIPE + q * CHUNK, CHUNK)],
                sems[q % 2])
        for q in range(nst - 2, nst):
            stores[q].wait()

    return scatter


def _combine_body(gf_ref, t0_ref, t1_ref, t2_ref, t3_ref, out_ref):
    gf = gf_ref[...]
    t = (t0_ref[...] + t1_ref[...]) + (t2_ref[...] + t3_ref[...])
    sums = t[:, :D]
    counts = t[:, D:D + 1]
    upd = BETA * gf + (1.0 - BETA) * sums / jnp.maximum(counts, 1.0)
    g = jnp.where(counts > 0, upd, gf)
    n = jnp.sqrt(jnp.sum(g * g, axis=1, keepdims=True))
    out_ref[...] = g / jnp.maximum(n, 1e-12)


_COMBINE_NBLK = 8


def _combine(gf, table_a, table_b):
    nblk = _COMBINE_NBLK
    blk = NG // nblk
    return pl.pallas_call(
        _combine_body,
        grid=(nblk,),
        in_specs=[
            pl.BlockSpec((blk, D), lambda i: (i, 0)),
            pl.BlockSpec((blk, WIDTH), lambda i: (i, 0)),
            pl.BlockSpec((blk, WIDTH), lambda i: (i + nblk, 0)),
            pl.BlockSpec((blk, WIDTH), lambda i: (i, 0)),
            pl.BlockSpec((blk, WIDTH), lambda i: (i + nblk, 0)),
        ],
        out_specs=pl.BlockSpec((blk, D), lambda i: (i, 0)),
        out_shape=jax.ShapeDtypeStruct((NG, D), jnp.float32),
    )(gf, table_a, table_a, table_b, table_b)


def kernel(x, group_features):
    rows_a = (3 * NS) // 4
    asn_a, xaug_a = _assign(x[:rows_a], group_features)
    asn_b, xaug_b = _assign(x[rows_a:], group_features)
    nca = rows_a // (CHUNK * NW)
    ncb = (NS - rows_a) // (CHUNK * NW)
    table_a = _make_scatter(nca)(xaug_a, asn_a.reshape(NW, nca, CHUNK))
    table_b = _make_scatter(ncb)(xaug_b, asn_b.reshape(NW, ncb, CHUNK))
    return _combine(group_features, table_a, table_b)


# speedup vs baseline: 1.0266x; 1.0266x over previous
import functools

import jax
import jax.numpy as jnp
from jax import lax
from jax.experimental import pallas as pl
from jax.experimental.pallas import tpu as pltpu
from jax.experimental.pallas import tpu_sc as plsc

NG = 8192
D = 32
NS = 16384
BETA = 0.99
WIDTH = 128
XB = 512
NW = 32
ROWS_PER_W = NS // NW
CHUNK = 128
NCHUNK = ROWS_PER_W // CHUNK
STRIPE = NG // 16


GB = 128
NGC = NG // GB


def _assign_body(x_ref, gf_ref, out_ref, xaug_ref, gfn_ref):
    i = pl.program_id(0)

    @pl.when(i == 0)
    def _():
        gf = gf_ref[...]
        nn = jnp.sqrt(jnp.sum(gf * gf, axis=1, keepdims=True))
        gfn_ref[...] = gf / jnp.maximum(nn, 1e-12)

    x = x_ref[...]
    xaug_ref[...] = jnp.concatenate(
        [x, jnp.ones((XB, 1), jnp.float32),
         jnp.zeros((XB, WIDTH - D - 1), jnp.float32)], axis=1)
    n = jnp.sqrt(jnp.sum(x * x, axis=1, keepdims=True))
    xn = x / jnp.maximum(n, 1e-12)

    m_run = jnp.full((XB, GB), -jnp.inf, jnp.float32)
    c_run = jnp.zeros((XB, GB), jnp.float32)
    for c in range(NGC):
        chunk = lax.dot_general(
            xn, gfn_ref[c * GB:(c + 1) * GB, :], (((1,), (1,)), ((), ())),
            preferred_element_type=jnp.float32)
        better = chunk > m_run
        c_run = jnp.where(better, jnp.float32(c), c_run)
        m_run = jnp.maximum(chunk, m_run)

    m = jnp.max(m_run, axis=-1, keepdims=True)
    lane = lax.broadcasted_iota(jnp.int32, (XB, GB), 1).astype(jnp.float32)
    j = c_run * GB + lane
    loc = jnp.min(jnp.where(m_run == m, j, jnp.float32(NG)), axis=-1)
    out_ref[0] = loc.astype(jnp.int32).reshape(XB // CHUNK, CHUNK)


def _assign(x, gf):
    grid = NS // XB
    rpb = XB // CHUNK
    return pl.pallas_call(
        _assign_body,
        grid=(grid,),
        in_specs=[
            pl.BlockSpec((XB, D), lambda i: (i, 0)),
            pl.BlockSpec((NG, D), lambda i: (0, 0)),
        ],
        out_specs=[
            pl.BlockSpec((1, rpb, CHUNK), lambda i: (i, 0, 0)),
            pl.BlockSpec((XB, WIDTH), lambda i: (i, 0)),
        ],
        out_shape=[
            jax.ShapeDtypeStruct((grid, rpb, CHUNK), jnp.int32),
            jax.ShapeDtypeStruct((NS, WIDTH), jnp.float32),
        ],
        scratch_shapes=[pltpu.VMEM((NG, D), jnp.float32)],
    )(x, gf)


@functools.lru_cache(maxsize=1)
def _make_scatter():
    mesh = plsc.VectorSubcoreMesh(core_axis_name="c", subcore_axis_name="s")

    @functools.partial(
        pl.kernel,
        mesh=mesh,
        out_type=jax.ShapeDtypeStruct((2 * NG, WIDTH), jnp.float32),
        scratch_types=[
            pltpu.VMEM((NCHUNK, CHUNK), jnp.int32),
            pltpu.VMEM((CHUNK, WIDTH), jnp.float32),
            pltpu.VMEM((CHUNK, WIDTH), jnp.float32),
            pltpu.SemaphoreType.DMA,
            pltpu.SemaphoreType.DMA,
            pltpu.VMEM_SHARED((NG, WIDTH), jnp.float32),
        ],
    )
    def scatter(xaug_hbm, asn_hbm, out_hbm, idx_v, buf0, buf1, sem0, sem1,
                table_sh):
        c = lax.axis_index("c")
        s = lax.axis_index("s")
        wid = s * 2 + c
        base = wid * ROWS_PER_W
        bufs = (buf0, buf1)
        sems = (sem0, sem1)

        zv = jnp.zeros((16,), jnp.float32)

        def zrow(r, carry):
            for k in range(WIDTH // 16):
                buf0[r, pl.ds(k * 16, 16)] = zv
            return carry

        lax.fori_loop(0, CHUNK, zrow, 0)
        for q in range(STRIPE // CHUNK):
            pltpu.sync_copy(buf0, table_sh.at[pl.ds(s * STRIPE + q * CHUNK, CHUNK)])
        pltpu.sync_copy(asn_hbm.at[wid], idx_v)
        plsc.subcore_barrier()

        copies = [None] * NCHUNK
        copies[0] = pltpu.async_copy(
            xaug_hbm.at[pl.ds(base, CHUNK)], bufs[0], sems[0])
        for q in range(NCHUNK):
            copies[q].wait()
            if q + 1 < NCHUNK:
                copies[q + 1] = pltpu.async_copy(
                    xaug_hbm.at[pl.ds(base + (q + 1) * CHUNK, CHUNK)],
                    bufs[(q + 1) % 2], sems[(q + 1) % 2])
            pltpu.sync_copy(bufs[q % 2], table_sh.at[idx_v.at[q]], add=True)
        plsc.subcore_barrier()

        nst = STRIPE // CHUNK
        stores = [None] * nst
        for q in range(nst):
            if q >= 2:
                stores[q - 2].wait()
            pltpu.sync_copy(table_sh.at[pl.ds(s * STRIPE + q * CHUNK, CHUNK)],
                            bufs[q % 2])
            stores[q] = pltpu.async_copy(
                bufs[q % 2],
                out_hbm.at[pl.ds(c * NG + s * STRIPE + q * CHUNK, CHUNK)],
                sems[q % 2])
        for q in range(nst - 2, nst):
            stores[q].wait()

    return scatter


def _combine_body(gf_ref, t0_ref, t1_ref, out_ref):
    gf = gf_ref[...]
    t = t0_ref[...] + t1_ref[...]
    sums = t[:, :D]
    counts = t[:, D:D + 1]
    upd = BETA * gf + (1.0 - BETA) * sums / jnp.maximum(counts, 1.0)
    g = jnp.where(counts > 0, upd, gf)
    n = jnp.sqrt(jnp.sum(g * g, axis=1, keepdims=True))
    out_ref[...] = g / jnp.maximum(n, 1e-12)


_COMBINE_NBLK = 8


def _combine(gf, table):
    nblk = _COMBINE_NBLK
    blk = NG // nblk
    return pl.pallas_call(
        _combine_body,
        grid=(nblk,),
        in_specs=[
            pl.BlockSpec((blk, D), lambda i: (i, 0)),
            pl.BlockSpec((blk, WIDTH), lambda i: (i, 0)),
            pl.BlockSpec((blk, WIDTH), lambda i: (i + nblk, 0)),
        ],
        out_specs=pl.BlockSpec((blk, D), lambda i: (i, 0)),
        out_shape=jax.ShapeDtypeStruct((NG, D), jnp.float32),
    )(gf, table, table)


def kernel(x, group_features):
    asn, xaug = _assign(x, group_features)
    table = _make_scatter()(xaug, asn)
    return _combine(group_features, table)
